# baseline (speedup 1.0000x reference)
import functools

import jax
import jax.numpy as jnp
from jax import lax
from jax.experimental import pallas as pl
from jax.experimental.pallas import tpu as pltpu
from jax.experimental.pallas import tpu_sc as plsc

D = 64
VOCAB = 1000000
NFULL = VOCAB // 128
TAIL = VOCAB - NFULL * 128
BATCH = 4096
HIST = 200
NJ = BATCH // 128


def _iota16():
    return lax.iota(jnp.int32, 16)


def _make_transpose_kernel():
    info = plsc.get_sparse_core_info()
    nw = info.num_cores * info.num_subcores
    mesh = plsc.VectorSubcoreMesh(core_axis_name="c", subcore_axis_name="s")

    @functools.partial(
        pl.kernel,
        mesh=mesh,
        out_type=jax.ShapeDtypeStruct((VOCAB // 2, 128), jnp.float32),
        scratch_types=[
            pltpu.VMEM((2, 64, 128), jnp.float32),
            pltpu.VMEM((2, 64, 128), jnp.float32),
            pltpu.VMEM((64, 128), jnp.float32),
        ]
        + [pltpu.SemaphoreType.DMA] * 4,
        compiler_params=pltpu.CompilerParams(
            use_tc_tiling_on_sc=True, needs_layout_passes=False
        ),
    )
    def kern_a(tabT_hbm, tailT_hbm, out_hbm, tin, tloc, swz, gs0, gs1, os0, os1):
        gs = (gs0, gs1)
        osm = (os0, os1)
        wid = lax.axis_index("s") * info.num_cores + lax.axis_index("c")

        def blk(k):
            return wid + nw * k

        nk = lax.select(wid < (NFULL % nw), (NFULL // nw) + 1, NFULL // nw)

        def start_in(j, b):
            pltpu.make_async_copy(
                tabT_hbm.at[:, pl.ds(j * 128, 128)], tin.at[b], gs[b]
            ).start()

        def wait_in(b):
            pltpu.make_async_copy(
                tabT_hbm.at[:, pl.ds(0, 128)], tin.at[b], gs[b]
            ).wait()

        def start_out(j, b):
            pltpu.make_async_copy(
                tloc.at[b], out_hbm.at[pl.ds(j * 64, 64)], osm[b]
            ).start()

        def wait_out(b):
            pltpu.make_async_copy(
                tloc.at[b], out_hbm.at[pl.ds(0, 64)], osm[b]
            ).wait()

        def transpose_block(b, np_=64):
            @plsc.parallel_loop(0, 64, unroll=16)
            def restage(r):
                for u in range(8):
                    v = tin[b, r, pl.ds(16 * u, 16)]
                    cols = (_iota16() + (16 * u + r)) & 127
                    plsc.store_scatter(
                        swz, [jnp.full((16,), r, dtype=jnp.int32), cols], v
                    )

            @plsc.parallel_loop(0, np_, unroll=16)
            def emit(p):
                for t in range(8):
                    rows = _iota16() + (16 * (t % 4))
                    cols = (rows + (2 * p + (t // 4))) & 127
                    v = plsc.load_gather(swz, [rows, cols])
                    tloc[b, p, pl.ds(16 * t, 16)] = v

        npairs = (nk + 1) // 2

        @pl.when(nk > 0)
        def _():
            start_in(blk(0), 0)

            def pair(i, carry):
                for b in range(2):
                    k = 2 * i + b

                    @pl.when(k < nk)
                    def _():
                        wait_in(b)

                        @pl.when(k + 1 < nk)
                        def _():
                            start_in(blk(k + 1), 1 - b)

                        @pl.when(k >= 2)
                        def _():
                            wait_out(b)

                        transpose_block(b)
                        start_out(blk(k), b)

                return carry

            lax.fori_loop(0, npairs, pair, 0)
            wait_out(0)
            wait_out(1)

        @pl.when(wid == 0)
        def _():
            pltpu.sync_copy(tailT_hbm, tin.at[0])

            def body(p, carry):
                for t in range(8):
                    rows = _iota16() + (16 * (t % 4))
                    cols = jnp.full((16,), 2 * p + (t // 4), dtype=jnp.int32)
                    v = plsc.load_gather(tin.at[0], [rows, cols])
                    tloc[0, p, pl.ds(16 * t, 16)] = v
                return carry

            lax.fori_loop(0, 32, body, 0)
            pltpu.sync_copy(
                tloc.at[0, pl.ds(0, 32)],
                out_hbm.at[pl.ds(NFULL * 64, 32)],
            )

    return kern_a


def _make_gather_kernel():
    info = plsc.get_sparse_core_info()
    nw = info.num_cores * info.num_subcores
    assert nw == NJ
    mesh = plsc.VectorSubcoreMesh(core_axis_name="c", subcore_axis_name="s")

    @functools.partial(
        pl.kernel,
        mesh=mesh,
        out_type=jax.ShapeDtypeStruct((HIST, 8, NJ, 8, 128), jnp.float32),
        scratch_types=[
            pltpu.VMEM((HIST, 128), jnp.int32),
            pltpu.VMEM((2, 128, D), jnp.float32),
            pltpu.VMEM((2, 8, 8, 128), jnp.float32),
            pltpu.VMEM((128, D), jnp.float32),
        ]
        + [pltpu.SemaphoreType.DMA] * 4,
        compiler_params=pltpu.CompilerParams(
            use_tc_tiling_on_sc=False, needs_layout_passes=False
        ),
    )
    def kern_bc(tab_hbm, xt_hbm, out_hbm, idxv, rows, tout, swz,
                gs0, gs1, os0, os1):
        gs = (gs0, gs1)
        osm = (os0, os1)
        wid = lax.axis_index("s") * info.num_cores + lax.axis_index("c")

        pltpu.sync_copy(xt_hbm.at[:, pl.ds(wid * 128, 128)], idxv)

        def start_gather(h, b):
            pltpu.make_async_copy(
                tab_hbm.at[idxv.at[h]], rows.at[b], gs[b]
            ).start()

        def wait_gather(b):
            pltpu.make_async_copy(
                tab_hbm.at[pl.ds(0, 128)], rows.at[b], gs[b]
            ).wait()

        def start_write(h, b):
            pltpu.make_async_copy(
                tout.at[b], out_hbm.at[h, :, wid], osm[b]
            ).start()

        def wait_write(b):
            pltpu.make_async_copy(
                tout.at[0], out_hbm.at[0, :, 0], osm[b]
            ).wait()

        def transpose_unit(b):
            @plsc.parallel_loop(0, 128, unroll=16)
            def restage(l):
                for u in range(4):
                    v = rows[b, l, pl.ds(16 * u, 16)]
                    cols = (_iota16() + (16 * u + l)) & 63
                    plsc.store_scatter(
                        swz, [jnp.full((16,), l, dtype=jnp.int32), cols], v
                    )

            @plsc.parallel_loop(0, D, unroll=16)
            def emit(d):
                for t in range(8):
                    rws = _iota16() + 16 * t
                    cls = (rws + d) & 63
                    v = plsc.load_gather(swz, [rws, cls])
                    tout[b, d // 8, d % 8, pl.ds(16 * t, 16)] = v

        start_gather(0, 0)
        start_gather(1, 1)

        def pair(i, carry):
            for b in range(2):
                h = 2 * i + b
                wait_gather(b)

                @pl.when(i >= 1)
                def _():
                    wait_write(b)

                transpose_unit(b)
                start_write(h, b)

                @pl.when(h + 2 < HIST)
                def _():
                    start_gather(h + 2, b)

            return carry

        lax.fori_loop(0, HIST // 2, pair, 0)
        wait_write(0)
        wait_write(1)

    return kern_bc


def kernel(x, table):
    tabT = table.T
    tailT = jnp.pad(table[VOCAB - TAIL :].T, ((0, 0), (0, 128 - TAIL)))
    lin = _make_transpose_kernel()(tabT, tailT)
    tab_lin = lin.reshape(VOCAB, D)
    xt = x.T.astype(jnp.int32)
    out5 = _make_gather_kernel()(tab_lin, xt)
    return out5.transpose((2, 4, 0, 1, 3)).reshape(BATCH, HIST, D)


# speedup vs baseline: 1.0062x; 1.0062x over previous
import functools

import jax
import jax.numpy as jnp
from jax import lax
from jax.experimental import pallas as pl
from jax.experimental.pallas import tpu as pltpu
from jax.experimental.pallas import tpu_sc as plsc

D = 64
VOCAB = 1000000
NFULL = VOCAB // 128
TAIL = VOCAB - NFULL * 128
BATCH = 4096
HIST = 200
NJ = BATCH // 128


def _iota16():
    return lax.iota(jnp.int32, 16)


def _make_transpose_kernel():
    info = plsc.get_sparse_core_info()
    nw = info.num_cores * info.num_subcores
    mesh = plsc.VectorSubcoreMesh(core_axis_name="c", subcore_axis_name="s")

    @functools.partial(
        pl.kernel,
        mesh=mesh,
        out_type=jax.ShapeDtypeStruct((VOCAB // 2, 128), jnp.float32),
        scratch_types=[
            pltpu.VMEM((2, 64, 128), jnp.float32),
            pltpu.VMEM((2, 64, 128), jnp.float32),
            pltpu.VMEM((64, 128), jnp.float32),
        ]
        + [pltpu.SemaphoreType.DMA] * 4,
        compiler_params=pltpu.CompilerParams(
            use_tc_tiling_on_sc=True, needs_layout_passes=False
        ),
    )
    def kern_a(tabT_hbm, tailT_hbm, out_hbm, tin, tloc, swz, gs0, gs1, os0, os1):
        gs = (gs0, gs1)
        osm = (os0, os1)
        wid = lax.axis_index("s") * info.num_cores + lax.axis_index("c")

        def blk(k):
            return wid + nw * k

        nk = lax.select(wid < (NFULL % nw), (NFULL // nw) + 1, NFULL // nw)

        def start_in(j, b):
            pltpu.make_async_copy(
                tabT_hbm.at[:, pl.ds(j * 128, 128)], tin.at[b], gs[b]
            ).start()

        def wait_in(b):
            pltpu.make_async_copy(
                tabT_hbm.at[:, pl.ds(0, 128)], tin.at[b], gs[b]
            ).wait()

        def start_out(j, b):
            pltpu.make_async_copy(
                tloc.at[b], out_hbm.at[pl.ds(j * 64, 64)], osm[b]
            ).start()

        def wait_out(b):
            pltpu.make_async_copy(
                tloc.at[b], out_hbm.at[pl.ds(0, 64)], osm[b]
            ).wait()

        def transpose_block(b, np_=64):
            @plsc.parallel_loop(0, 64, unroll=8)
            def restage(r):
                for u in range(8):
                    v = tin[b, r, pl.ds(16 * u, 16)]
                    cols = (_iota16() + (16 * u + r)) & 127
                    plsc.store_scatter(
                        swz, [jnp.full((16,), r, dtype=jnp.int32), cols], v
                    )

            @plsc.parallel_loop(0, np_, unroll=8)
            def emit(p):
                for t in range(8):
                    rows = _iota16() + (16 * (t % 4))
                    cols = (rows + (2 * p + (t // 4))) & 127
                    v = plsc.load_gather(swz, [rows, cols])
                    tloc[b, p, pl.ds(16 * t, 16)] = v

        npairs = (nk + 1) // 2

        @pl.when(nk > 0)
        def _():
            start_in(blk(0), 0)

            def pair(i, carry):
                for b in range(2):
                    k = 2 * i + b

                    @pl.when(k < nk)
                    def _():
                        wait_in(b)

                        @pl.when(k + 1 < nk)
                        def _():
                            start_in(blk(k + 1), 1 - b)

                        @pl.when(k >= 2)
                        def _():
                            wait_out(b)

                        transpose_block(b)
                        start_out(blk(k), b)

                return carry

            lax.fori_loop(0, npairs, pair, 0)
            wait_out(0)
            wait_out(1)

        @pl.when(wid == 0)
        def _():
            pltpu.sync_copy(tailT_hbm, tin.at[0])

            def body(p, carry):
                for t in range(8):
                    rows = _iota16() + (16 * (t % 4))
                    cols = jnp.full((16,), 2 * p + (t // 4), dtype=jnp.int32)
                    v = plsc.load_gather(tin.at[0], [rows, cols])
                    tloc[0, p, pl.ds(16 * t, 16)] = v
                return carry

            lax.fori_loop(0, 32, body, 0)
            pltpu.sync_copy(
                tloc.at[0, pl.ds(0, 32)],
                out_hbm.at[pl.ds(NFULL * 64, 32)],
            )

    return kern_a


def _make_gather_kernel():
    info = plsc.get_sparse_core_info()
    nw = info.num_cores * info.num_subcores
    assert nw == NJ
    mesh = plsc.VectorSubcoreMesh(core_axis_name="c", subcore_axis_name="s")

    @functools.partial(
        pl.kernel,
        mesh=mesh,
        out_type=jax.ShapeDtypeStruct((HIST, 8, NJ, 8, 128), jnp.float32),
        scratch_types=[
            pltpu.VMEM((HIST, 128), jnp.int32),
            pltpu.VMEM((2, 128, D), jnp.float32),
            pltpu.VMEM((2, 8, 8, 128), jnp.float32),
            pltpu.VMEM((128, D), jnp.float32),
        ]
        + [pltpu.SemaphoreType.DMA] * 4,
        compiler_params=pltpu.CompilerParams(
            use_tc_tiling_on_sc=False, needs_layout_passes=False
        ),
    )
    def kern_bc(tab_hbm, xt_hbm, out_hbm, idxv, rows, tout, swz,
                gs0, gs1, os0, os1):
        gs = (gs0, gs1)
        osm = (os0, os1)
        wid = lax.axis_index("s") * info.num_cores + lax.axis_index("c")

        pltpu.sync_copy(xt_hbm.at[:, pl.ds(wid * 128, 128)], idxv)

        def start_gather(h, b):
            pltpu.make_async_copy(
                tab_hbm.at[idxv.at[h]], rows.at[b], gs[b]
            ).start()

        def wait_gather(b):
            pltpu.make_async_copy(
                tab_hbm.at[pl.ds(0, 128)], rows.at[b], gs[b]
            ).wait()

        def start_write(h, b):
            pltpu.make_async_copy(
                tout.at[b], out_hbm.at[h, :, wid], osm[b]
            ).start()

        def wait_write(b):
            pltpu.make_async_copy(
                tout.at[0], out_hbm.at[0, :, 0], osm[b]
            ).wait()

        def transpose_unit(b):
            @plsc.parallel_loop(0, 128, unroll=8)
            def restage(l):
                for u in range(4):
                    v = rows[b, l, pl.ds(16 * u, 16)]
                    cols = (_iota16() + (16 * u + l)) & 63
                    plsc.store_scatter(
                        swz, [jnp.full((16,), l, dtype=jnp.int32), cols], v
                    )

            @plsc.parallel_loop(0, D, unroll=8)
            def emit(d):
                for t in range(8):
                    rws = _iota16() + 16 * t
                    cls = (rws + d) & 63
                    v = plsc.load_gather(swz, [rws, cls])
                    tout[b, d // 8, d % 8, pl.ds(16 * t, 16)] = v

        start_gather(0, 0)
        start_gather(1, 1)

        def pair(i, carry):
            for b in range(2):
                h = 2 * i + b
                wait_gather(b)

                @pl.when(i >= 1)
                def _():
                    wait_write(b)

                transpose_unit(b)
                start_write(h, b)

                @pl.when(h + 2 < HIST)
                def _():
                    start_gather(h + 2, b)

            return carry

        lax.fori_loop(0, HIST // 2, pair, 0)
        wait_write(0)
        wait_write(1)

    return kern_bc


def kernel(x, table):
    tabT = table.T
    tailT = jnp.pad(table[VOCAB - TAIL :].T, ((0, 0), (0, 128 - TAIL)))
    lin = _make_transpose_kernel()(tabT, tailT)
    tab_lin = lin.reshape(VOCAB, D)
    xt = x.T.astype(jnp.int32)
    out5 = _make_gather_kernel()(tab_lin, xt)
    return out5.transpose((2, 4, 0, 1, 3)).reshape(BATCH, HIST, D)
